# Initial kernel scaffold; baseline (speedup 1.0000x reference)
import jax
import jax.numpy as jnp
from jax.experimental import pallas as pl


def kernel(x, edge_index, xgb_scores, Wl1, bl1, Wr1, g1, b1, Wl2, bl2, Wr2, g2, b2, Wl3, bl3, Wr3, g3, b3, Fw1, Fb1, Fw2, Fb2):
    raise NotImplementedError("write your pallas kernel here")


# speedup vs baseline: 5.7390x; 5.7390x over previous
import functools

import jax
import jax.numpy as jnp
from jax import lax
from jax.experimental import pallas as pl
from jax.experimental.pallas import tpu as pltpu
from jax.experimental.pallas import tpu_sc as plsc

_N = 10000
_E = 320000
_D = 128
_CHUNK = 128
_NCHUNKS = _E // _CHUNK
_NC = 2
_NS = 16
_NW = _NC * _NS
_RPT = 640
_STEPS = -(-_NCHUNKS // _NW)


def _tile_chunks(s):
    if s < _NS - 1:
        return [(k * 128, 128) for k in range(5)]
    return [(0, 128), (128, 128), (256, 128), (384, 16)]


def _make_sc_agg(with_cnt):
    mesh = plsc.VectorSubcoreMesh(core_axis_name="c", subcore_axis_name="s")
    outs = [jax.ShapeDtypeStruct((_NC, _N, _D), jnp.float32)]
    scratch = [
        pltpu.VMEM((_CHUNK,), jnp.int32),
        pltpu.VMEM((1, _CHUNK), jnp.int32),
        pltpu.VMEM((_CHUNK, _D), jnp.float32),
        pltpu.VMEM_SHARED((_N, _D), jnp.float32),
        pltpu.SemaphoreType.DMA,
    ]
    if with_cnt:
        outs.append(jax.ShapeDtypeStruct((_NC, _N, 16), jnp.float32))
        scratch += [
            pltpu.VMEM((_CHUNK, 16), jnp.float32),
            pltpu.VMEM((_CHUNK, 16), jnp.float32),
            pltpu.VMEM_SHARED((_N, 16), jnp.float32),
        ]

    def body(h_hbm, src_hbm, dst_hbm, z_hbm, *rest):
        if with_cnt:
            (z16_hbm, ones_hbm, agg_out, cnt_out,
             sidx, didx, rows, acc, gsem, ones_v, cbuf, cacc) = rest
        else:
            agg_out, sidx, didx, rows, acc, gsem = rest
        c = lax.axis_index("c")
        s = lax.axis_index("s")
        w = s * _NC + c
        r0 = s * _RPT

        def _per_chunk(fn):
            @pl.when(s < _NS - 1)
            def _():
                for o, n in _tile_chunks(0):
                    fn(r0 + o, n)

            @pl.when(s == _NS - 1)
            def _():
                for o, n in _tile_chunks(_NS - 1):
                    fn(r0 + o, n)

        pltpu.sync_copy(z_hbm.at[pl.ds(0, _CHUNK)], rows)
        _per_chunk(lambda o, n: pltpu.sync_copy(rows.at[pl.ds(0, n)],
                                                acc.at[pl.ds(o, n)]))
        if with_cnt:
            pltpu.sync_copy(z16_hbm.at[pl.ds(0, _CHUNK)], cbuf)
            _per_chunk(lambda o, n: pltpu.sync_copy(cbuf.at[pl.ds(0, n)],
                                                    cacc.at[pl.ds(o, n)]))
            pltpu.sync_copy(ones_hbm, ones_v)
        plsc.subcore_barrier()

        def step(j, carry):
            cidx = w + j * _NW

            @pl.when(cidx < _NCHUNKS)
            def _():
                pltpu.sync_copy(src_hbm.at[pl.ds(cidx * _CHUNK, _CHUNK)], sidx)
                pltpu.sync_copy(dst_hbm.at[cidx], didx)
                pltpu.async_copy(h_hbm.at[sidx], rows, gsem).wait()
                pltpu.sync_copy(rows, acc.at[didx.at[0]], add=True)
                if with_cnt:
                    pltpu.sync_copy(ones_v, cacc.at[didx.at[0]], add=True)

            return carry

        lax.fori_loop(0, _STEPS, step, 0)
        plsc.subcore_barrier()
        def _wb(o, n):
            pltpu.sync_copy(acc.at[pl.ds(o, n)], rows.at[pl.ds(0, n)])
            pltpu.sync_copy(rows.at[pl.ds(0, n)], agg_out.at[c, pl.ds(o, n)])

        _per_chunk(_wb)
        if with_cnt:
            def _wbc(o, n):
                pltpu.sync_copy(cacc.at[pl.ds(o, n)], cbuf.at[pl.ds(0, n)])
                pltpu.sync_copy(cbuf.at[pl.ds(0, n)], cnt_out.at[c, pl.ds(o, n)])

            _per_chunk(_wbc)

    return pl.kernel(body, out_type=tuple(outs) if with_cnt else outs[0],
                     mesh=mesh, scratch_types=scratch)


_sc_agg = _make_sc_agg(False)


def _make_sc_cnt():
    mesh = plsc.VectorSubcoreMesh(core_axis_name="c", subcore_axis_name="s")
    scratch = [
        pltpu.VMEM((1, _CHUNK), jnp.int32),
        pltpu.VMEM((_CHUNK, _D), jnp.float32),
        pltpu.VMEM_SHARED((_N, _D), jnp.float32),
    ]

    def body(dst_hbm, z_hbm, ones_hbm, cnt_out, didx, buf, cacc):
        c = lax.axis_index("c")
        s = lax.axis_index("s")
        w = s * _NC + c
        r0 = s * _RPT

        def _per_chunk(fn):
            @pl.when(s < _NS - 1)
            def _():
                for o, n in _tile_chunks(0):
                    fn(r0 + o, n)

            @pl.when(s == _NS - 1)
            def _():
                for o, n in _tile_chunks(_NS - 1):
                    fn(r0 + o, n)

        pltpu.sync_copy(z_hbm.at[pl.ds(0, _CHUNK)], buf)
        _per_chunk(lambda o, n: pltpu.sync_copy(buf.at[pl.ds(0, n)],
                                                cacc.at[pl.ds(o, n)]))
        pltpu.sync_copy(ones_hbm, buf)
        plsc.subcore_barrier()

        def step(j, carry):
            cidx = w + j * _NW

            @pl.when(cidx < _NCHUNKS)
            def _():
                pltpu.sync_copy(dst_hbm.at[cidx], didx)
                pltpu.sync_copy(buf, cacc.at[didx.at[0]], add=True)

            return carry

        lax.fori_loop(0, _STEPS, step, 0)
        plsc.subcore_barrier()

        def _wbc(o, n):
            pltpu.sync_copy(cacc.at[pl.ds(o, n)], buf.at[pl.ds(0, n)])
            pltpu.sync_copy(buf.at[pl.ds(0, n)], cnt_out.at[c, pl.ds(o, n)])

        _per_chunk(_wbc)

    return pl.kernel(body, out_type=jax.ShapeDtypeStruct((_NC, _N, _D), jnp.float32),
                     mesh=mesh, scratch_types=scratch)


_sc_cnt = _make_sc_cnt()


def _mmT(a, b):
    return lax.dot_general(a, b, (((1,), (1,)), ((), ())),
                           preferred_element_type=jnp.float32)


def _bn_relu(pre, g, b):
    mu = jnp.mean(pre, axis=0, keepdims=True)
    var = jnp.mean((pre - mu) ** 2, axis=0, keepdims=True)
    return jnp.maximum((pre - mu) * lax.rsqrt(var + 1e-5) * g[None, :]
                       + b[None, :], 0.0)


def _sage_pre(aggp, cntp, h, wl, bl, wr):
    cnt = cntp[0][:, 0:1] + cntp[1][:, 0:1]
    mean = (aggp[0] + aggp[1]) * (1.0 / jnp.maximum(cnt, 1.0))
    return _mmT(mean, wl[...]) + bl[...][None, :] + _mmT(h[...], wr[...])


def _dense_body(aggp, cntp, h, wl, bl, wr, g, b, out):
    pre = _sage_pre(aggp, cntp, h, wl, bl, wr)
    out[...] = _bn_relu(pre, g[...], b[...])


def _final_body(aggp, cntp, h, wl, bl, wr, g, b, xgb8, fw1a, fw1x8, fb1,
                fw2p, fb2p, out):
    h3 = _bn_relu(_sage_pre(aggp, cntp, h, wl, bl, wr), g[...], b[...])
    z = jnp.maximum(_mmT(h3, fw1a[...]) + _mmT(xgb8[...], fw1x8[...])
                    + fb1[...][None, :], 0.0)
    out[...] = _mmT(z, fw2p[...]) + fb2p[...][None, :]


_dense = pl.pallas_call(
    _dense_body, out_shape=jax.ShapeDtypeStruct((_N, _D), jnp.float32))
_final = pl.pallas_call(
    _final_body, out_shape=jax.ShapeDtypeStruct((_N, 8), jnp.float32))


def kernel(x, edge_index, xgb_scores, Wl1, bl1, Wr1, g1, b1, Wl2, bl2, Wr2,
           g2, b2, Wl3, bl3, Wr3, g3, b3, Fw1, Fb1, Fw2, Fb2):
    src1d = edge_index[0]
    dst3d = edge_index[1].reshape(_NCHUNKS, 1, _CHUNK)
    zeros = jnp.zeros((_N, _D), jnp.float32)
    ones128 = jnp.ones((_CHUNK, _D), jnp.float32)
    cntp = _sc_cnt(dst3d, zeros, ones128)[:, :, 0:16]
    aggp = _sc_agg(x, src1d, dst3d, zeros)
    h1 = _dense(aggp, cntp, x, Wl1, bl1, Wr1, g1, b1)
    aggp = _sc_agg(h1, src1d, dst3d, zeros)
    h2 = _dense(aggp, cntp, h1, Wl2, bl2, Wr2, g2, b2)
    aggp = _sc_agg(h2, src1d, dst3d, zeros)
    xgb8 = jnp.pad(xgb_scores[:, None], ((0, 0), (0, 7)))
    fw1x8 = jnp.pad(Fw1[:, _D:_D + 1], ((0, 0), (0, 7)))
    fw2p = jnp.pad(Fw2, ((0, 7), (0, 0)))
    fb2p = jnp.pad(Fb2, (0, 7))
    out = _final(aggp, cntp, h2, Wl3, bl3, Wr3, g3, b3, xgb8, Fw1[:, 0:_D],
                 fw1x8, Fb1, fw2p, fb2p)
    return out[:, 0]


# speedup vs baseline: 10.8289x; 1.8869x over previous
import jax
import jax.numpy as jnp
from jax import lax
from jax.experimental import pallas as pl
from jax.experimental.pallas import tpu as pltpu
from jax.experimental.pallas import tpu_sc as plsc

_N = 10000
_E = 320000
_D = 128
_CHUNK = 128
_ECR = 1
_EC = _ECR * _CHUNK
_NC = 2
_NS = 16
_NW = _NC * _NS
_EPT = 9984
_CPT = _EPT // _EC
_EMAIN = _EPT * _NW
_TAIL = (_E - _EMAIN) // _NW
_NB = 5
_RPT = 640


def _tile_chunks(s):
    if s < _NS - 1:
        return [(k * 128, 128) for k in range(5)]
    return [(0, 128), (128, 128), (256, 128), (384, 16)]


def _mk_per_chunk(s, r0):
    def _per_chunk(fn):
        @pl.when(s < _NS - 1)
        def _():
            for o, n in _tile_chunks(0):
                fn(r0 + o, n)

        @pl.when(s == _NS - 1)
        def _():
            for o, n in _tile_chunks(_NS - 1):
                fn(r0 + o, n)

    return _per_chunk


def _make_sc_agg():
    mesh = plsc.VectorSubcoreMesh(core_axis_name="c", subcore_axis_name="s")
    scratch = [
        pltpu.VMEM((_CPT, _ECR, _CHUNK), jnp.int32),
        pltpu.VMEM((1, _TAIL), jnp.int32),
        pltpu.VMEM((_TAIL,), jnp.int32),
        pltpu.VMEM_SHARED((_N, _D), jnp.float32),
    ]
    scratch += [pltpu.VMEM((_EC, _D), jnp.float32) for _ in range(2)]
    scratch += [pltpu.VMEM((1, _CHUNK), jnp.int32) for _ in range(2)]
    scratch += [pltpu.SemaphoreType.DMA for _ in range(6)]

    def body(h_hbm, src4_hbm, dst4_hbm, stail_hbm, dtail_hbm, z_hbm, agg_out,
             sidx, dtl, stl, acc, r0b, r1b, d0b, d1b, g0, g1, s0, s1, d0, d1):
        rows = [r0b, r1b]
        didx = [d0b, d1b]
        gsem = [g0, g1]
        ssem = [s0, s1]
        dsem = [d0, d1]
        c = lax.axis_index("c")
        s = lax.axis_index("s")
        w = s * _NC + c
        r0 = s * _RPT
        per_chunk = _mk_per_chunk(s, r0)

        pltpu.sync_copy(src4_hbm.at[pl.ds(w * _CPT, _CPT)], sidx)
        pltpu.sync_copy(stail_hbm.at[pl.ds(w * _TAIL, _TAIL)], stl)
        pltpu.sync_copy(dtail_hbm.at[w], dtl)
        pltpu.sync_copy(z_hbm.at[pl.ds(0, _CHUNK)], rows[0])
        per_chunk(lambda o, n: pltpu.sync_copy(rows[0].at[pl.ds(0, n)],
                                               acc.at[pl.ds(o, n)]))
        plsc.subcore_barrier()

        base = w * _CPT

        def _dfire(j, b):
            pltpu.async_copy(dst4_hbm.at[base + j], didx[b], dsem[b])

        def _dwait(b):
            pltpu.make_async_copy(dst4_hbm.at[base], didx[b], dsem[b]).wait()

        def _gfire(j, b):
            pltpu.async_copy(h_hbm.at[sidx.at[j, 0]], rows[b], gsem[b])

        def _gwait(b):
            pltpu.make_async_copy(h_hbm.at[sidx.at[0, 0]], rows[b], gsem[b]).wait()

        def _sfire(j, b):
            pltpu.async_copy(rows[b], acc.at[didx[b].at[0]], ssem[b], add=True)

        def _sdrain(b):
            pltpu.make_async_copy(h_hbm.at[sidx.at[0, 0]], rows[b], ssem[b]).wait()

        def group(g, carry):
            for k in range(2):
                j = g * 2 + k
                b = k
                b2 = (k + 1) % 2

                @pl.when(jnp.logical_and(j >= 2, j - 2 < _CPT))
                def _():
                    _sdrain(b)

                @pl.when(j < _CPT)
                def _():
                    _gfire(j, b)

                @pl.when(j < _CPT)
                def _():
                    _dfire(j, b)

                jj = j - 1

                @pl.when(jnp.logical_and(jj >= 0, jj < _CPT))
                def _():
                    _dwait(b2)
                    _gwait(b2)
                    _sfire(jj, b2)

            return carry

        lax.fori_loop(0, _CPT // 2 + 2, group, 0)
        pltpu.async_copy(h_hbm.at[stl], rows[0].at[pl.ds(0, _TAIL)], gsem[0]).wait()
        pltpu.sync_copy(rows[0].at[pl.ds(0, _TAIL)], acc.at[dtl.at[0]], add=True)
        plsc.subcore_barrier()

        def _wb(o, n):
            pltpu.sync_copy(acc.at[pl.ds(o, n)], rows[0].at[pl.ds(0, n)])
            pltpu.sync_copy(rows[0].at[pl.ds(0, n)], agg_out.at[c, pl.ds(o, n)])

        per_chunk(_wb)

    return pl.kernel(body, out_type=jax.ShapeDtypeStruct((_NC, _N, _D), jnp.float32),
                     mesh=mesh, scratch_types=scratch)


_sc_agg = _make_sc_agg()


def _make_sc_cnt():
    mesh = plsc.VectorSubcoreMesh(core_axis_name="c", subcore_axis_name="s")
    scratch = [
        pltpu.VMEM((_CPT, _ECR, _CHUNK), jnp.int32),
        pltpu.VMEM((1, _TAIL), jnp.int32),
        pltpu.VMEM((_EC, _D), jnp.float32),
        pltpu.VMEM_SHARED((_N, _D), jnp.float32),
    ]
    scratch += [pltpu.SemaphoreType.DMA for _ in range(_NB)]

    def body(dst4_hbm, dtail_hbm, z_hbm, ones_hbm, cnt_out,
             didx, dtl, buf, cacc, *ssem):
        ssem = list(ssem)
        c = lax.axis_index("c")
        s = lax.axis_index("s")
        w = s * _NC + c
        r0 = s * _RPT
        per_chunk = _mk_per_chunk(s, r0)

        pltpu.sync_copy(dst4_hbm.at[pl.ds(w * _CPT, _CPT)], didx)
        pltpu.sync_copy(dtail_hbm.at[w], dtl)
        pltpu.sync_copy(z_hbm.at[pl.ds(0, _CHUNK)], buf.at[pl.ds(0, _CHUNK)])
        per_chunk(lambda o, n: pltpu.sync_copy(buf.at[pl.ds(0, n)],
                                               cacc.at[pl.ds(o, n)]))
        pltpu.sync_copy(ones_hbm, buf)
        plsc.subcore_barrier()

        def _sfire(j, b):
            pltpu.async_copy(buf, cacc.at[didx.at[j, 0]], ssem[b], add=True)

        def _sdrain(b):
            pltpu.make_async_copy(z_hbm.at[pl.ds(0, _EC)], buf, ssem[b]).wait()

        def group(g, carry):
            for k in range(_NB):
                j = g * _NB + k

                @pl.when(jnp.logical_and(j >= _NB, j - _NB < _CPT))
                def _():
                    _sdrain(k)

                @pl.when(j < _CPT)
                def _():
                    _sfire(j, k)

            return carry

        lax.fori_loop(0, (_CPT + _NB - 1) // _NB + 1, group, 0)
        pltpu.sync_copy(buf.at[pl.ds(0, _TAIL)], cacc.at[dtl.at[0]], add=True)
        plsc.subcore_barrier()

        def _wbc(o, n):
            pltpu.sync_copy(cacc.at[pl.ds(o, n)], buf.at[pl.ds(0, n)])
            pltpu.sync_copy(buf.at[pl.ds(0, n)], cnt_out.at[c, pl.ds(o, n)])

        per_chunk(_wbc)

    return pl.kernel(body, out_type=jax.ShapeDtypeStruct((_NC, _N, _D), jnp.float32),
                     mesh=mesh, scratch_types=scratch)


_sc_cnt = _make_sc_cnt()


def _mmT(a, b):
    return lax.dot_general(a, b, (((1,), (1,)), ((), ())),
                           preferred_element_type=jnp.float32)


def _bn_relu(pre, g, b):
    mu = jnp.mean(pre, axis=0, keepdims=True)
    var = jnp.mean((pre - mu) ** 2, axis=0, keepdims=True)
    return jnp.maximum((pre - mu) * lax.rsqrt(var + 1e-5) * g[None, :]
                       + b[None, :], 0.0)


def _sage_pre(aggp, cntp, h, wl, bl, wr):
    cnt = cntp[0][:, 0:1] + cntp[1][:, 0:1]
    mean = (aggp[0] + aggp[1]) * (1.0 / jnp.maximum(cnt, 1.0))
    return _mmT(mean, wl[...]) + bl[...][None, :] + _mmT(h[...], wr[...])


def _dense_body(aggp, cntp, h, wl, bl, wr, g, b, out):
    pre = _sage_pre(aggp, cntp, h, wl, bl, wr)
    out[...] = _bn_relu(pre, g[...], b[...])


def _final_body(aggp, cntp, h, wl, bl, wr, g, b, xgb8, fw1a, fw1x8, fb1,
                fw2p, fb2p, out):
    h3 = _bn_relu(_sage_pre(aggp, cntp, h, wl, bl, wr), g[...], b[...])
    z = jnp.maximum(_mmT(h3, fw1a[...]) + _mmT(xgb8[...], fw1x8[...])
                    + fb1[...][None, :], 0.0)
    out[...] = _mmT(z, fw2p[...]) + fb2p[...][None, :]


_dense = pl.pallas_call(
    _dense_body, out_shape=jax.ShapeDtypeStruct((_N, _D), jnp.float32))
_final = pl.pallas_call(
    _final_body, out_shape=jax.ShapeDtypeStruct((_N, 8), jnp.float32))


def kernel(x, edge_index, xgb_scores, Wl1, bl1, Wr1, g1, b1, Wl2, bl2, Wr2,
           g2, b2, Wl3, bl3, Wr3, g3, b3, Fw1, Fb1, Fw2, Fb2):
    src = edge_index[0]
    dst = edge_index[1]
    src4 = src[:_EMAIN].reshape(_EMAIN // _EC, _ECR, _CHUNK)
    dst4 = dst[:_EMAIN].reshape(_EMAIN // _EC, _ECR, _CHUNK)
    stail = src[_EMAIN:]
    dtail3 = dst[_EMAIN:].reshape(_NW, 1, _TAIL)
    zeros = jnp.zeros((_N, _D), jnp.float32)
    ones128 = jnp.ones((_EC, _D), jnp.float32)
    cntp = _sc_cnt(dst4, dtail3, zeros, ones128)[:, :, 0:16]
    aggp = _sc_agg(x, src4, dst4, stail, dtail3, zeros)
    h1 = _dense(aggp, cntp, x, Wl1, bl1, Wr1, g1, b1)
    aggp = _sc_agg(h1, src4, dst4, stail, dtail3, zeros)
    h2 = _dense(aggp, cntp, h1, Wl2, bl2, Wr2, g2, b2)
    aggp = _sc_agg(h2, src4, dst4, stail, dtail3, zeros)
    xgb8 = jnp.pad(xgb_scores[:, None], ((0, 0), (0, 7)))
    fw1x8 = jnp.pad(Fw1[:, _D:_D + 1], ((0, 0), (0, 7)))
    fw2p = jnp.pad(Fw2, ((0, 7), (0, 0)))
    fb2p = jnp.pad(Fb2, (0, 7))
    out = _final(aggp, cntp, h2, Wl3, bl3, Wr3, g3, b3, xgb8, Fw1[:, 0:_D],
                 fw1x8, Fb1, fw2p, fb2p)
    return out[:, 0]


# speedup vs baseline: 11.4235x; 1.0549x over previous
import jax
import jax.numpy as jnp
from jax import lax
from jax.experimental import pallas as pl
from jax.experimental.pallas import tpu as pltpu
from jax.experimental.pallas import tpu_sc as plsc

_N = 10000
_E = 320000
_D = 128
_CHUNK = 128
_ECR = 1
_EC = _ECR * _CHUNK
_NC = 2
_NS = 16
_NW = _NC * _NS
_EPT = 9984
_CPT = _EPT // _EC
_EMAIN = _EPT * _NW
_TAIL = (_E - _EMAIN) // _NW
_NB = 5
_RPT = 640


def _tile_chunks(s):
    if s < _NS - 1:
        return [(k * 128, 128) for k in range(5)]
    return [(0, 128), (128, 128), (256, 128), (384, 16)]


def _mk_per_chunk(s, r0):
    def _per_chunk(fn):
        @pl.when(s < _NS - 1)
        def _():
            for o, n in _tile_chunks(0):
                fn(r0 + o, n)

        @pl.when(s == _NS - 1)
        def _():
            for o, n in _tile_chunks(_NS - 1):
                fn(r0 + o, n)

    return _per_chunk


def _make_sc_agg():
    mesh = plsc.VectorSubcoreMesh(core_axis_name="c", subcore_axis_name="s")
    scratch = [
        pltpu.VMEM((1, _TAIL), jnp.int32),
        pltpu.VMEM((_TAIL,), jnp.int32),
        pltpu.VMEM_SHARED((_N, _D), jnp.float32),
    ]
    scratch += [pltpu.VMEM((_EC, _D), jnp.float32) for _ in range(3)]
    scratch += [pltpu.VMEM((1, _CHUNK), jnp.int32) for _ in range(3)]
    scratch += [pltpu.VMEM((1, _CHUNK), jnp.int32) for _ in range(4)]
    scratch += [pltpu.SemaphoreType.DMA for _ in range(13)]

    def body(h_hbm, src4_hbm, dst4_hbm, stail_hbm, dtail_hbm, z_hbm, agg_out,
             dtl, stl, acc, *rest):
        rows = list(rest[0:3])
        sidx = list(rest[3:6])
        didx = list(rest[6:10])
        gsem = list(rest[10:13])
        ssem = list(rest[13:16])
        sisem = list(rest[16:19])
        disem = list(rest[19:23])
        c = lax.axis_index("c")
        s = lax.axis_index("s")
        w = s * _NC + c
        r0 = s * _RPT
        per_chunk = _mk_per_chunk(s, r0)

        pltpu.sync_copy(stail_hbm.at[pl.ds(w * _TAIL, _TAIL)], stl)
        pltpu.sync_copy(dtail_hbm.at[w], dtl)
        pltpu.sync_copy(z_hbm.at[pl.ds(0, _CHUNK)], rows[0])
        per_chunk(lambda o, n: pltpu.sync_copy(rows[0].at[pl.ds(0, n)],
                                               acc.at[pl.ds(o, n)]))
        plsc.subcore_barrier()

        base = w * _CPT

        def _sifire(j, m):
            pltpu.async_copy(src4_hbm.at[base + j], sidx[m], sisem[m])

        def _siwait(m):
            pltpu.make_async_copy(src4_hbm.at[base], sidx[m], sisem[m]).wait()

        def _dfire(j, m):
            pltpu.async_copy(dst4_hbm.at[base + j], didx[m], disem[m])

        def _dwait(m):
            pltpu.make_async_copy(dst4_hbm.at[base], didx[m], disem[m]).wait()

        def _gfire(j, b):
            pltpu.async_copy(h_hbm.at[sidx[b].at[0]], rows[b], gsem[b])

        def _gwait(b):
            pltpu.make_async_copy(h_hbm.at[sidx[0].at[0]], rows[b], gsem[b]).wait()

        def _sfire(j, b, m):
            pltpu.async_copy(rows[b], acc.at[didx[m].at[0]], ssem[b], add=True)

        def _sdrain(b):
            pltpu.make_async_copy(h_hbm.at[sidx[0].at[0]], rows[b], ssem[b]).wait()

        _sifire(0, 0)
        _dfire(0, 0)

        def group(g, carry):
            for k in range(12):
                j = g * 12 + k
                b = k % 3

                @pl.when(jnp.logical_and(j >= 3, j - 3 < _CPT))
                def _():
                    _sdrain(b)

                @pl.when(j < _CPT)
                def _():
                    _siwait(b)
                    _gfire(j, b)

                jn = j + 1
                mn = (k + 1) % 3
                dn = (k + 1) % 4

                @pl.when(jn < _CPT)
                def _():
                    _sifire(jn, mn)
                    _dfire(jn, dn)

                jj = j - 1
                b2 = (k - 1) % 3
                m2 = (k - 1) % 4

                @pl.when(jnp.logical_and(jj >= 0, jj < _CPT))
                def _():
                    _dwait(m2)
                    _gwait(b2)
                    _sfire(jj, b2, m2)

            return carry

        lax.fori_loop(0, (_CPT + 3 + 11) // 12 + 1, group, 0)
        pltpu.async_copy(h_hbm.at[stl], rows[0].at[pl.ds(0, _TAIL)], gsem[0]).wait()
        pltpu.sync_copy(rows[0].at[pl.ds(0, _TAIL)], acc.at[dtl.at[0]], add=True)
        plsc.subcore_barrier()

        def _wb(o, n):
            pltpu.sync_copy(acc.at[pl.ds(o, n)], rows[0].at[pl.ds(0, n)])
            pltpu.sync_copy(rows[0].at[pl.ds(0, n)], agg_out.at[c, pl.ds(o, n)])

        per_chunk(_wb)

    return pl.kernel(body, out_type=jax.ShapeDtypeStruct((_NC, _N, _D), jnp.float32),
                     mesh=mesh, scratch_types=scratch)


_sc_agg = _make_sc_agg()


def _make_sc_cnt():
    mesh = plsc.VectorSubcoreMesh(core_axis_name="c", subcore_axis_name="s")
    scratch = [
        pltpu.VMEM((_CPT, _ECR, _CHUNK), jnp.int32),
        pltpu.VMEM((1, _TAIL), jnp.int32),
        pltpu.VMEM((_EC, _D), jnp.float32),
        pltpu.VMEM_SHARED((_N, _D), jnp.float32),
    ]
    scratch += [pltpu.SemaphoreType.DMA for _ in range(_NB)]

    def body(dst4_hbm, dtail_hbm, z_hbm, ones_hbm, cnt_out,
             didx, dtl, buf, cacc, *ssem):
        ssem = list(ssem)
        c = lax.axis_index("c")
        s = lax.axis_index("s")
        w = s * _NC + c
        r0 = s * _RPT
        per_chunk = _mk_per_chunk(s, r0)

        pltpu.sync_copy(dst4_hbm.at[pl.ds(w * _CPT, _CPT)], didx)
        pltpu.sync_copy(dtail_hbm.at[w], dtl)
        pltpu.sync_copy(z_hbm.at[pl.ds(0, _CHUNK)], buf.at[pl.ds(0, _CHUNK)])
        per_chunk(lambda o, n: pltpu.sync_copy(buf.at[pl.ds(0, n)],
                                               cacc.at[pl.ds(o, n)]))
        pltpu.sync_copy(ones_hbm, buf)
        plsc.subcore_barrier()

        def _sfire(j, b):
            pltpu.async_copy(buf, cacc.at[didx.at[j, 0]], ssem[b], add=True)

        def _sdrain(b):
            pltpu.make_async_copy(z_hbm.at[pl.ds(0, _EC)], buf, ssem[b]).wait()

        def group(g, carry):
            for k in range(_NB):
                j = g * _NB + k

                @pl.when(jnp.logical_and(j >= _NB, j - _NB < _CPT))
                def _():
                    _sdrain(k)

                @pl.when(j < _CPT)
                def _():
                    _sfire(j, k)

            return carry

        lax.fori_loop(0, (_CPT + _NB - 1) // _NB + 1, group, 0)
        pltpu.sync_copy(buf.at[pl.ds(0, _TAIL)], cacc.at[dtl.at[0]], add=True)
        plsc.subcore_barrier()

        def _wbc(o, n):
            pltpu.sync_copy(cacc.at[pl.ds(o, n)], buf.at[pl.ds(0, n)])
            pltpu.sync_copy(buf.at[pl.ds(0, n)], cnt_out.at[c, pl.ds(o, n)])

        per_chunk(_wbc)

    return pl.kernel(body, out_type=jax.ShapeDtypeStruct((_NC, _N, _D), jnp.float32),
                     mesh=mesh, scratch_types=scratch)


_sc_cnt = _make_sc_cnt()


def _mmT(a, b):
    return lax.dot_general(a, b, (((1,), (1,)), ((), ())),
                           preferred_element_type=jnp.float32)


def _bn_relu(pre, g, b):
    mu = jnp.mean(pre, axis=0, keepdims=True)
    var = jnp.mean((pre - mu) ** 2, axis=0, keepdims=True)
    return jnp.maximum((pre - mu) * lax.rsqrt(var + 1e-5) * g[None, :]
                       + b[None, :], 0.0)


def _sage_pre(aggp, cntp, h, wl, bl, wr):
    cnt = cntp[0][:, 0:1] + cntp[1][:, 0:1]
    mean = (aggp[0] + aggp[1]) * (1.0 / jnp.maximum(cnt, 1.0))
    return _mmT(mean, wl[...]) + bl[...][None, :] + _mmT(h[...], wr[...])


def _dense_body(aggp, cntp, h, wl, bl, wr, g, b, out):
    pre = _sage_pre(aggp, cntp, h, wl, bl, wr)
    out[...] = _bn_relu(pre, g[...], b[...])


def _final_body(aggp, cntp, h, wl, bl, wr, g, b, xgb8, fw1a, fw1x8, fb1,
                fw2p, fb2p, out):
    h3 = _bn_relu(_sage_pre(aggp, cntp, h, wl, bl, wr), g[...], b[...])
    z = jnp.maximum(_mmT(h3, fw1a[...]) + _mmT(xgb8[...], fw1x8[...])
                    + fb1[...][None, :], 0.0)
    out[...] = _mmT(z, fw2p[...]) + fb2p[...][None, :]


_dense = pl.pallas_call(
    _dense_body, out_shape=jax.ShapeDtypeStruct((_N, _D), jnp.float32))
_final = pl.pallas_call(
    _final_body, out_shape=jax.ShapeDtypeStruct((_N, 8), jnp.float32))


def kernel(x, edge_index, xgb_scores, Wl1, bl1, Wr1, g1, b1, Wl2, bl2, Wr2,
           g2, b2, Wl3, bl3, Wr3, g3, b3, Fw1, Fb1, Fw2, Fb2):
    src = edge_index[0]
    dst = edge_index[1]
    src4 = src[:_EMAIN].reshape(_EMAIN // _EC, _ECR, _CHUNK)
    dst4 = dst[:_EMAIN].reshape(_EMAIN // _EC, _ECR, _CHUNK)
    stail = src[_EMAIN:]
    dtail3 = dst[_EMAIN:].reshape(_NW, 1, _TAIL)
    zeros = jnp.zeros((_N, _D), jnp.float32)
    ones128 = jnp.ones((_EC, _D), jnp.float32)
    cntp = _sc_cnt(dst4, dtail3, zeros, ones128)[:, :, 0:16]
    aggp = _sc_agg(x, src4, dst4, stail, dtail3, zeros)
    h1 = _dense(aggp, cntp, x, Wl1, bl1, Wr1, g1, b1)
    aggp = _sc_agg(h1, src4, dst4, stail, dtail3, zeros)
    h2 = _dense(aggp, cntp, h1, Wl2, bl2, Wr2, g2, b2)
    aggp = _sc_agg(h2, src4, dst4, stail, dtail3, zeros)
    xgb8 = jnp.pad(xgb_scores[:, None], ((0, 0), (0, 7)))
    fw1x8 = jnp.pad(Fw1[:, _D:_D + 1], ((0, 0), (0, 7)))
    fw2p = jnp.pad(Fw2, ((0, 7), (0, 0)))
    fb2p = jnp.pad(Fb2, (0, 7))
    out = _final(aggp, cntp, h2, Wl3, bl3, Wr3, g3, b3, xgb8, Fw1[:, 0:_D],
                 fw1x8, Fb1, fw2p, fb2p)
    return out[:, 0]
